# Initial kernel scaffold; baseline (speedup 1.0000x reference)
import functools
import numpy as np
import jax
import jax.numpy as jnp
from jax import lax
from jax.experimental import pallas as pl
from jax.experimental.pallas import tpu as pltpu
from jax.experimental.pallas import tpu_sc as plsc

N = 10000
E = 320000
D = 128
D_OUT = 64
R = 8
VOCAB = 100000

NC, NS, L = 2, 16, 16
NW = NC * NS

EB = 128
NBLK = 79
E_PAD = NW * NBLK * EB
EPT = NBLK * EB

XBLK = 3
X_PAD = NW * XBLK * EB

C_NODES = 2000
ROWS = C_NODES * R
NPASS = 5
ROWS_PT = ROWS // NS
ZROWS = 500

CN_PAD = N * R + 8

_MESH = plsc.VectorSubcoreMesh(
    core_axis_name="c", subcore_axis_name="s", num_cores=NC, num_subcores=NS
)


def _wid():
    return lax.axis_index("s") * NC + lax.axis_index("c")


def _emb_body(emb_hbm, x_hbm, h_hbm, idx_v, rows_v, sem):
    w = _wid()
    for j in range(XBLK):
        base = (w * XBLK + j) * EB
        pltpu.sync_copy(x_hbm.at[pl.ds(base, EB)], idx_v)
        pltpu.async_copy(emb_hbm.at[idx_v], rows_v, sem).wait()
        pltpu.sync_copy(rows_v, h_hbm.at[pl.ds(base, EB)])


def _emb_gather(emb, x_pad):
    return pl.kernel(
        _emb_body,
        out_type=jax.ShapeDtypeStruct((X_PAD, D), jnp.float32),
        mesh=_MESH,
        scratch_types=[
            pltpu.VMEM((EB,), jnp.int32),
            pltpu.VMEM((EB, D), jnp.float32),
            pltpu.SemaphoreType.DMA,
        ],
    )(emb, x_pad)


def _cnt_body(dst_hbm, typ_hbm, zc_hbm, cnt_hbm, d_v, t_v, acc_v):
    w = _wid()
    pltpu.sync_copy(zc_hbm, acc_v)
    ones = jnp.ones((L,), jnp.float32)

    def blk(i, _):
        base = w * EPT + i * EB
        pltpu.sync_copy(dst_hbm.at[pl.ds(base, EB)], d_v)
        pltpu.sync_copy(typ_hbm.at[pl.ds(base, EB)], t_v)
        for j in range(EB // L):
            d = d_v[pl.ds(j * L, L)]
            t = t_v[pl.ds(j * L, L)]
            plsc.addupdate_scatter(acc_v, [d * R + t], ones)
        return 0

    lax.fori_loop(0, NBLK, blk, 0)
    pltpu.sync_copy(acc_v, cnt_hbm.at[w])


def _edge_counts(dst_pad, typ_pad):
    zc = jnp.zeros((CN_PAD,), jnp.float32)
    return pl.kernel(
        _cnt_body,
        out_type=jax.ShapeDtypeStruct((NW, CN_PAD), jnp.float32),
        mesh=_MESH,
        scratch_types=[
            pltpu.VMEM((EB,), jnp.int32),
            pltpu.VMEM((EB,), jnp.int32),
            pltpu.VMEM((CN_PAD,), jnp.float32),
        ],
    )(dst_pad, typ_pad, zc)


def _agg_body(h_hbm, src_hbm, dst_hbm, typ_hbm, a_hbm,
              src_v, dst_v, typ_v, key_v, rows_v, zero_v, acc_sp, sem):
    c = lax.axis_index("c")
    s = lax.axis_index("s")
    w = s * NC + c
    zz = jnp.zeros((L,), jnp.float32)
    for jj in range(D // L):
        zero_v[0, pl.ds(jj * L, L)] = zz
    for p in range(NPASS):
        for z in range(ROWS_PT // ZROWS):
            pltpu.sync_copy(
                zero_v, acc_sp.at[pl.ds(s * ROWS_PT + z * ZROWS, ZROWS)]
            )
        plsc.subcore_barrier()
        lo = p * ROWS

        def blk(i, _):
            base = w * EPT + i * EB
            pltpu.sync_copy(src_hbm.at[pl.ds(base, EB)], src_v)
            pltpu.sync_copy(dst_hbm.at[pl.ds(base, EB)], dst_v)
            pltpu.sync_copy(typ_hbm.at[pl.ds(base, EB)], typ_v)
            for j in range(EB // L):
                d = dst_v[pl.ds(j * L, L)]
                t = typ_v[pl.ds(j * L, L)]
                g = d * R + t - lo
                inr = (g >= 0) & (g < ROWS)
                key_v[pl.ds(j * L, L)] = jnp.where(inr, g, ROWS)
            pltpu.async_copy(h_hbm.at[src_v], rows_v, sem).wait()
            pltpu.sync_copy(rows_v, acc_sp.at[key_v], add=True)
            return 0

        lax.fori_loop(0, NBLK, blk, 0)
        plsc.subcore_barrier()
        pltpu.sync_copy(
            acc_sp.at[pl.ds(s * ROWS_PT, ROWS_PT)],
            a_hbm.at[c].at[pl.ds(lo + s * ROWS_PT, ROWS_PT)],
        )


def _edge_aggregate(h_tab, src_pad, dst_pad, typ_pad):
    zrows = jnp.zeros((ZROWS, D), jnp.float32)
    return pl.kernel(
        _agg_body,
        out_type=jax.ShapeDtypeStruct((NC, N * R, D), jnp.float32),
        mesh=_MESH,
        scratch_types=[
            pltpu.VMEM((EB,), jnp.int32),
            pltpu.VMEM((EB,), jnp.int32),
            pltpu.VMEM((EB,), jnp.int32),
            pltpu.VMEM((EB,), jnp.int32),
            pltpu.VMEM((EB, D), jnp.float32),
            pltpu.VMEM((ZROWS, D), jnp.float32),
            pltpu.VMEM_SHARED((ROWS + 8, D), jnp.float32),
            pltpu.SemaphoreType.DMA,
        ],
    )(h_tab, src_pad, dst_pad, typ_pad)


NB = 500
GRID = N // NB


def _layer_math(h_ref, a_ref, cnt_ref, w_ref, b_ref):
    c = jnp.sum(cnt_ref[...], axis=0)
    inv = 1.0 / jnp.maximum(c, 1.0)
    acc = jnp.dot(h_ref[...], w_ref[0:D, :],
                  preferred_element_type=jnp.float32) + b_ref[0, :][None, :]
    for r in range(R):
        ar = (a_ref[0, :, r, :] + a_ref[1, :, r, :]) * inv[:, r][:, None]
        acc = acc + jnp.dot(ar, w_ref[D * (r + 1):D * (r + 2), :],
                            preferred_element_type=jnp.float32)
    return jnp.maximum(acc, 0.0)


def _layer_body(h_ref, a_ref, cnt_ref, w_ref, b_ref, out_ref):
    out_ref[...] = _layer_math(h_ref, a_ref, cnt_ref, w_ref, b_ref)


def _tc_layer(h, a4, cnt, w_cat, bias, d_out):
    return pl.pallas_call(
        _layer_body,
        grid=(GRID,),
        in_specs=[
            pl.BlockSpec((NB, D), lambda i: (i, 0)),
            pl.BlockSpec((NC, NB, R, D), lambda i: (0, i, 0, 0)),
            pl.BlockSpec((NW, NB, R), lambda i: (0, i, 0)),
            pl.BlockSpec(((R + 1) * D, d_out), lambda i: (0, 0)),
            pl.BlockSpec((1, d_out), lambda i: (0, 0)),
        ],
        out_specs=pl.BlockSpec((NB, d_out), lambda i: (i, 0)),
        out_shape=jax.ShapeDtypeStruct((N, d_out), jnp.float32),
    )(h, a4, cnt, w_cat, bias)


def _pool_matrix():
    idx = np.arange(D_OUT)
    starts = (idx * N) // D_OUT
    ends = ((idx + 1) * N + D_OUT - 1) // D_OUT
    p = np.zeros((D_OUT, N), np.float32)
    for b in range(D_OUT):
        p[b, starts[b]:ends[b]] = 1.0 / float(ends[b] - starts[b])
    return jnp.asarray(p)


def _layer2_body(h_ref, a_ref, cnt_ref, w_ref, b_ref, p_ref, out_ref):
    h3 = _layer_math(h_ref, a_ref, cnt_ref, w_ref, b_ref)

    @pl.when(pl.program_id(0) == 0)
    def _():
        out_ref[...] = jnp.zeros_like(out_ref)

    out_ref[...] += jnp.dot(p_ref[...], h3, preferred_element_type=jnp.float32)


def _tc_layer2_pool(h, a4, cnt, w_cat, bias, pmat):
    return pl.pallas_call(
        _layer2_body,
        grid=(GRID,),
        in_specs=[
            pl.BlockSpec((NB, D), lambda i: (i, 0)),
            pl.BlockSpec((NC, NB, R, D), lambda i: (0, i, 0, 0)),
            pl.BlockSpec((NW, NB, R), lambda i: (0, i, 0)),
            pl.BlockSpec(((R + 1) * D, D_OUT), lambda i: (0, 0)),
            pl.BlockSpec((1, D_OUT), lambda i: (0, 0)),
            pl.BlockSpec((D_OUT, NB), lambda i: (0, i)),
        ],
        out_specs=pl.BlockSpec((D_OUT, D_OUT), lambda i: (0, 0)),
        out_shape=jax.ShapeDtypeStruct((D_OUT, D_OUT), jnp.float32),
    )(h, a4, cnt, w_cat, bias, pmat)


def kernel(x, edge_index, edge_type, emb, w1, root1, b1, w2, root2, b2):
    src = edge_index[0]
    dst = edge_index[1]
    pad = E_PAD - E
    src_p = jnp.concatenate([src, jnp.zeros((pad,), jnp.int32)])
    dst_p = jnp.concatenate([dst, jnp.full((pad,), N, jnp.int32)])
    typ_p = jnp.concatenate([edge_type, jnp.zeros((pad,), jnp.int32)])
    x_p = jnp.concatenate([x, jnp.zeros((X_PAD - N,), jnp.int32)])

    w1c = jnp.concatenate([root1, w1.reshape(R * D, D)], axis=0)
    w2c = jnp.concatenate([root2, w2.reshape(R * D, D_OUT)], axis=0)

    h = _emb_gather(emb, x_p)
    cnt = _edge_counts(dst_p, typ_p)
    cnt = cnt[:, :N * R].reshape(NW, N, R)

    a1 = _edge_aggregate(h, src_p, dst_p, typ_p)
    a1 = a1.reshape(NC, N, R, D)
    h2 = _tc_layer(h[:N], a1, cnt, w1c, b1.reshape(1, D), D)

    a2 = _edge_aggregate(h2, src_p, dst_p, typ_p)
    a2 = a2.reshape(NC, N, R, D)
    return _tc_layer2_pool(h2, a2, cnt, w2c, b2.reshape(1, D_OUT),
                           _pool_matrix())


# speedup vs baseline: 1.3500x; 1.3500x over previous
import functools
import numpy as np
import jax
import jax.numpy as jnp
from jax import lax
from jax.experimental import pallas as pl
from jax.experimental.pallas import tpu as pltpu
from jax.experimental.pallas import tpu_sc as plsc

N = 10000
N_PAD = 10240
E = 320000
D = 128
D_OUT = 64
R = 8
VOCAB = 100000

NC, NS, L = 2, 16, 16
NW = NC * NS

EB = 128
NBLK = 79
E_PAD = NW * NBLK * EB
EPT = NBLK * EB

XBLK = 3
X_PAD = NW * XBLK * EB

C_NODES = 896
ROWS = C_NODES * R
NPASS = 12
AROWS_TOT = NPASS * ROWS
ROWS_PT = ROWS // NS
ZROWS = 448
CR_PT = 456

_MESH = plsc.VectorSubcoreMesh(
    core_axis_name="c", subcore_axis_name="s", num_cores=NC, num_subcores=NS
)


def _wid():
    return lax.axis_index("s") * NC + lax.axis_index("c")


def _emb_body(emb_hbm, x_hbm, h_hbm, idx_v, rows_v, sem):
    w = _wid()
    for j in range(XBLK):
        base = (w * XBLK + j) * EB
        pltpu.sync_copy(x_hbm.at[pl.ds(base, EB)], idx_v)
        pltpu.async_copy(emb_hbm.at[idx_v], rows_v, sem).wait()
        pltpu.sync_copy(rows_v, h_hbm.at[pl.ds(base, EB)])


def _emb_gather(emb, x_pad):
    return pl.kernel(
        _emb_body,
        out_type=jax.ShapeDtypeStruct((X_PAD, D), jnp.float32),
        mesh=_MESH,
        compiler_params=pltpu.CompilerParams(use_tc_tiling_on_sc=False),
        scratch_types=[
            pltpu.VMEM((EB,), jnp.int32),
            pltpu.VMEM((EB, D), jnp.float32),
            pltpu.SemaphoreType.DMA,
        ],
    )(emb, x_pad)


def _agg_body(h_hbm, src_hbm, dst_hbm, typ_hbm, z_hbm, zc_hbm, ones_hbm,
              a_hbm, cnt_hbm,
              src_v, dst_v, typ_v, key_v, rows_v, ones_v, acc_sp, cnt_sp, sem):
    c = lax.axis_index("c")
    s = lax.axis_index("s")
    w = s * NC + c
    pltpu.sync_copy(ones_hbm, ones_v)
    for p in range(NPASS):
        for z in range(ROWS_PT // ZROWS):
            pltpu.sync_copy(
                z_hbm, acc_sp.at[pl.ds(s * ROWS_PT + z * ZROWS, ZROWS)]
            )
        pltpu.sync_copy(zc_hbm, cnt_sp.at[pl.ds(s * CR_PT, CR_PT)])
        plsc.subcore_barrier()
        lo = p * ROWS

        def blk(i, _):
            base = w * EPT + i * EB
            pltpu.sync_copy(src_hbm.at[pl.ds(base, EB)], src_v)
            pltpu.sync_copy(dst_hbm.at[pl.ds(base, EB)], dst_v)
            pltpu.sync_copy(typ_hbm.at[pl.ds(base, EB)], typ_v)
            for j in range(EB // L):
                d = dst_v[pl.ds(j * L, L)]
                t = typ_v[pl.ds(j * L, L)]
                g = d * R + t - lo
                inr = (g >= 0) & (g < ROWS)
                key_v[pl.ds(j * L, L)] = jnp.where(inr, g, ROWS)
            pltpu.async_copy(h_hbm.at[src_v], rows_v, sem).wait()
            pltpu.sync_copy(rows_v, acc_sp.at[key_v], add=True)
            pltpu.sync_copy(ones_v, cnt_sp.at[key_v], add=True)
            return 0

        lax.fori_loop(0, NBLK, blk, 0)
        plsc.subcore_barrier()
        pltpu.sync_copy(
            acc_sp.at[pl.ds(s * ROWS_PT, ROWS_PT)],
            a_hbm.at[c].at[pl.ds(lo + s * ROWS_PT, ROWS_PT)],
        )
        pltpu.sync_copy(
            cnt_sp.at[pl.ds(s * ROWS_PT, ROWS_PT)],
            cnt_hbm.at[c].at[pl.ds(lo + s * ROWS_PT, ROWS_PT)],
        )
        plsc.subcore_barrier()


def _edge_aggregate(h_tab, src_pad, dst_pad, typ_pad):
    zrows = jnp.zeros((ZROWS, D), jnp.float32)
    zcnt = jnp.zeros((CR_PT, L), jnp.float32)
    ones = jnp.zeros((EB, L), jnp.float32).at[:, 0].set(1.0)
    return pl.kernel(
        _agg_body,
        out_type=(
            jax.ShapeDtypeStruct((NC, AROWS_TOT, D), jnp.float32),
            jax.ShapeDtypeStruct((NC, AROWS_TOT, L), jnp.float32),
        ),
        mesh=_MESH,
        compiler_params=pltpu.CompilerParams(use_tc_tiling_on_sc=False),
        scratch_types=[
            pltpu.VMEM((EB,), jnp.int32),
            pltpu.VMEM((EB,), jnp.int32),
            pltpu.VMEM((EB,), jnp.int32),
            pltpu.VMEM((EB,), jnp.int32),
            pltpu.VMEM((EB, D), jnp.float32),
            pltpu.VMEM((EB, L), jnp.float32),
            pltpu.VMEM_SHARED((ROWS + 8, D), jnp.float32),
            pltpu.VMEM_SHARED((NS * CR_PT, L), jnp.float32),
            pltpu.SemaphoreType.DMA,
        ],
    )(h_tab, src_pad, dst_pad, typ_pad, zrows, zcnt, ones)


NB = 512
GRID = N_PAD // NB


def _layer_math(h_ref, a_ref, cnt_ref, w_ref, b_ref):
    c = cnt_ref[0, :, :, 0] + cnt_ref[1, :, :, 0]
    inv = 1.0 / jnp.maximum(c, 1.0)
    acc = jnp.dot(h_ref[...], w_ref[0:D, :],
                  preferred_element_type=jnp.float32) + b_ref[0, :][None, :]
    for r in range(R):
        ar = (a_ref[0, :, r, :] + a_ref[1, :, r, :]) * inv[:, r][:, None]
        acc = acc + jnp.dot(ar, w_ref[D * (r + 1):D * (r + 2), :],
                            preferred_element_type=jnp.float32)
    return jnp.maximum(acc, 0.0)


def _layer_body(h_ref, a_ref, cnt_ref, w_ref, b_ref, out_ref):
    out_ref[...] = _layer_math(h_ref, a_ref, cnt_ref, w_ref, b_ref)


def _tc_layer(h, a4, cnt, w_cat, bias, d_out):
    return pl.pallas_call(
        _layer_body,
        grid=(GRID,),
        in_specs=[
            pl.BlockSpec((NB, D), lambda i: (i, 0)),
            pl.BlockSpec((NC, NB, R, D), lambda i: (0, i, 0, 0)),
            pl.BlockSpec((NC, NB, R, L), lambda i: (0, i, 0, 0)),
            pl.BlockSpec(((R + 1) * D, d_out), lambda i: (0, 0)),
            pl.BlockSpec((1, d_out), lambda i: (0, 0)),
        ],
        out_specs=pl.BlockSpec((NB, d_out), lambda i: (i, 0)),
        out_shape=jax.ShapeDtypeStruct((X_PAD, d_out), jnp.float32),
    )(h, a4, cnt, w_cat, bias)


def _pool_matrix():
    idx = np.arange(D_OUT)
    starts = (idx * N) // D_OUT
    ends = ((idx + 1) * N + D_OUT - 1) // D_OUT
    p = np.zeros((D_OUT, N_PAD), np.float32)
    for b in range(D_OUT):
        p[b, starts[b]:ends[b]] = 1.0 / float(ends[b] - starts[b])
    return jnp.asarray(p)


def _layer2_body(h_ref, a_ref, cnt_ref, w_ref, b_ref, p_ref, out_ref):
    h3 = _layer_math(h_ref, a_ref, cnt_ref, w_ref, b_ref)

    @pl.when(pl.program_id(0) == 0)
    def _():
        out_ref[...] = jnp.zeros_like(out_ref)

    out_ref[...] += jnp.dot(p_ref[...], h3, preferred_element_type=jnp.float32)


def _tc_layer2_pool(h, a4, cnt, w_cat, bias, pmat):
    return pl.pallas_call(
        _layer2_body,
        grid=(GRID,),
        in_specs=[
            pl.BlockSpec((NB, D), lambda i: (i, 0)),
            pl.BlockSpec((NC, NB, R, D), lambda i: (0, i, 0, 0)),
            pl.BlockSpec((NC, NB, R, L), lambda i: (0, i, 0, 0)),
            pl.BlockSpec(((R + 1) * D, D_OUT), lambda i: (0, 0)),
            pl.BlockSpec((1, D_OUT), lambda i: (0, 0)),
            pl.BlockSpec((D_OUT, NB), lambda i: (0, i)),
        ],
        out_specs=pl.BlockSpec((D_OUT, D_OUT), lambda i: (0, 0)),
        out_shape=jax.ShapeDtypeStruct((D_OUT, D_OUT), jnp.float32),
    )(h, a4, cnt, w_cat, bias, pmat)


def kernel(x, edge_index, edge_type, emb, w1, root1, b1, w2, root2, b2):
    src = edge_index[0]
    dst = edge_index[1]
    pad = E_PAD - E
    src_p = jnp.concatenate([src, jnp.zeros((pad,), jnp.int32)])
    dst_p = jnp.concatenate([dst, jnp.full((pad,), N_PAD, jnp.int32)])
    typ_p = jnp.concatenate([edge_type, jnp.zeros((pad,), jnp.int32)])
    x_p = jnp.concatenate([x, jnp.zeros((X_PAD - N,), jnp.int32)])

    w1c = jnp.concatenate([root1, w1.reshape(R * D, D)], axis=0)
    w2c = jnp.concatenate([root2, w2.reshape(R * D, D_OUT)], axis=0)

    h = _emb_gather(emb, x_p)

    a1, cnt = _edge_aggregate(h, src_p, dst_p, typ_p)
    a1 = a1.reshape(NC, AROWS_TOT // R, R, D)
    cnt = cnt.reshape(NC, AROWS_TOT // R, R, L)
    h2 = _tc_layer(h, a1, cnt, w1c, b1.reshape(1, D), D)

    a2, _ = _edge_aggregate(h2, src_p, dst_p, typ_p)
    a2 = a2.reshape(NC, AROWS_TOT // R, R, D)
    return _tc_layer2_pool(h2, a2, cnt, w2c, b2.reshape(1, D_OUT),
                           _pool_matrix())


# speedup vs baseline: 4.4859x; 3.3228x over previous
import functools
import numpy as np
import jax
import jax.numpy as jnp
from jax import lax
from jax.experimental import pallas as pl
from jax.experimental.pallas import tpu as pltpu
from jax.experimental.pallas import tpu_sc as plsc

N = 10000
N_PAD = 10240
E = 320000
D = 128
D_OUT = 64
R = 8
VOCAB = 100000

NC, NS, L = 2, 16, 16
NW = NC * NS

EB = 128
NBLK = 79
E_PAD = NW * NBLK * EB
EPT = NBLK * EB

XBLK = 3
X_PAD = NW * XBLK * EB

C_NODES = 896
ROWS = C_NODES * R
NPASS = 12
AROWS_TOT = NPASS * ROWS
ROWS_PT = ROWS // NS
ZROWS = 448
CR_PT = 456

_MESH = plsc.VectorSubcoreMesh(
    core_axis_name="c", subcore_axis_name="s", num_cores=NC, num_subcores=NS
)


def _wid():
    return lax.axis_index("s") * NC + lax.axis_index("c")


def _emb_body(emb_hbm, x_hbm, h_hbm, idx_v, rows_v, sem):
    w = _wid()
    for j in range(XBLK):
        base = (w * XBLK + j) * EB
        pltpu.sync_copy(x_hbm.at[pl.ds(base, EB)], idx_v)
        pltpu.async_copy(emb_hbm.at[idx_v], rows_v, sem).wait()
        pltpu.sync_copy(rows_v, h_hbm.at[pl.ds(base, EB)])


def _emb_gather(emb, x_pad):
    return pl.kernel(
        _emb_body,
        out_type=jax.ShapeDtypeStruct((X_PAD, D), jnp.float32),
        mesh=_MESH,
        compiler_params=pltpu.CompilerParams(
            use_tc_tiling_on_sc=False, needs_layout_passes=False),
        scratch_types=[
            pltpu.VMEM((EB,), jnp.int32),
            pltpu.VMEM((EB, D), jnp.float32),
            pltpu.SemaphoreType.DMA,
        ],
    )(emb, x_pad)


def _agg_body(h_hbm, src_hbm, dst_hbm, typ_hbm, z_hbm, zc_hbm, ones_hbm,
              a_hbm, cnt_hbm,
              e_src, e_key, c_src, c_key, src_row, key_row, rows_v, ones_v,
              acc_sp, cnt_sp, sem):
    c = lax.axis_index("c")
    s = lax.axis_index("s")
    w = s * NC + c
    pltpu.sync_copy(ones_hbm, ones_v)
    pltpu.sync_copy(src_hbm.at[pl.ds(w * EPT, EPT)], e_src)
    pltpu.sync_copy(dst_hbm.at[pl.ds(w * EPT, EPT)], e_key)
    pltpu.sync_copy(typ_hbm.at[pl.ds(w * EPT, EPT)],
                    c_key.at[pl.ds(0, EPT)])

    def mk(i, _):
        b = i * L
        e_key[pl.ds(b, L)] = e_key[pl.ds(b, L)] * R + c_key[pl.ds(b, L)]
        return 0

    lax.fori_loop(0, EPT // L, mk, 0)

    for p in range(NPASS):
        for z in range(ROWS_PT // ZROWS):
            pltpu.sync_copy(
                z_hbm, acc_sp.at[pl.ds(s * ROWS_PT + z * ZROWS, ZROWS)]
            )
        pltpu.sync_copy(zc_hbm, cnt_sp.at[pl.ds(s * CR_PT, CR_PT)])
        plsc.subcore_barrier()
        lo = p * ROWS

        def compact(i, off):
            b = i * L
            k = e_key[pl.ds(b, L)] - lo
            sv = e_src[pl.ds(b, L)]
            inr = (k >= 0) & (k < ROWS)
            plsc.store_compressed(c_key.at[pl.ds(off, L)], k, mask=inr)
            plsc.store_compressed(c_src.at[pl.ds(off, L)], sv, mask=inr)
            return off + jnp.sum(inr.astype(jnp.int32))

        off = lax.fori_loop(0, EPT // L, compact, jnp.int32(0))
        trash_k = jnp.full((L,), ROWS, jnp.int32)
        zero_s = jnp.zeros((L,), jnp.int32)
        for j in range(EB // L):
            c_key[pl.ds(off + j * L, L)] = trash_k
            c_src[pl.ds(off + j * L, L)] = zero_s
        nchunk = (off + EB - 1) // EB

        def chunk(q, _):
            b = q * EB
            for j in range(EB // L):
                src_row[pl.ds(j * L, L)] = c_src[pl.ds(b + j * L, L)]
                key_row[pl.ds(j * L, L)] = c_key[pl.ds(b + j * L, L)]
            pltpu.async_copy(h_hbm.at[src_row], rows_v, sem).wait()
            pltpu.sync_copy(rows_v, acc_sp.at[key_row], add=True)
            pltpu.sync_copy(ones_v, cnt_sp.at[key_row], add=True)
            return 0

        lax.fori_loop(0, nchunk, chunk, 0)
        plsc.subcore_barrier()
        pltpu.sync_copy(
            acc_sp.at[pl.ds(s * ROWS_PT, ROWS_PT)],
            a_hbm.at[c].at[pl.ds(lo + s * ROWS_PT, ROWS_PT)],
        )
        pltpu.sync_copy(
            cnt_sp.at[pl.ds(s * ROWS_PT, ROWS_PT)],
            cnt_hbm.at[c].at[pl.ds(lo + s * ROWS_PT, ROWS_PT)],
        )
        plsc.subcore_barrier()


def _edge_aggregate(h_tab, src_pad, dst_pad, typ_pad):
    zrows = jnp.zeros((ZROWS, D), jnp.float32)
    zcnt = jnp.zeros((CR_PT, L), jnp.float32)
    ones = jnp.zeros((EB, L), jnp.float32).at[:, 0].set(1.0)
    return pl.kernel(
        _agg_body,
        out_type=(
            jax.ShapeDtypeStruct((NC, AROWS_TOT, D), jnp.float32),
            jax.ShapeDtypeStruct((NC, AROWS_TOT, L), jnp.float32),
        ),
        mesh=_MESH,
        compiler_params=pltpu.CompilerParams(
            use_tc_tiling_on_sc=False, needs_layout_passes=False),
        scratch_types=[
            pltpu.VMEM((EPT,), jnp.int32),
            pltpu.VMEM((EPT,), jnp.int32),
            pltpu.VMEM((EPT + EB,), jnp.int32),
            pltpu.VMEM((EPT + EB,), jnp.int32),
            pltpu.VMEM((EB,), jnp.int32),
            pltpu.VMEM((EB,), jnp.int32),
            pltpu.VMEM((EB, D), jnp.float32),
            pltpu.VMEM((EB, L), jnp.float32),
            pltpu.VMEM_SHARED((ROWS + 8, D), jnp.float32),
            pltpu.VMEM_SHARED((NS * CR_PT, L), jnp.float32),
            pltpu.SemaphoreType.DMA,
        ],
    )(h_tab, src_pad, dst_pad, typ_pad, zrows, zcnt, ones)


NB = 512
GRID = N_PAD // NB


def _layer_math(h_ref, a_ref, cnt_ref, w_ref, b_ref):
    c = cnt_ref[0, :, :, 0] + cnt_ref[1, :, :, 0]
    inv = 1.0 / jnp.maximum(c, 1.0)
    acc = jnp.dot(h_ref[...], w_ref[0:D, :],
                  preferred_element_type=jnp.float32) + b_ref[0, :][None, :]
    for r in range(R):
        ar = (a_ref[0, :, r, :] + a_ref[1, :, r, :]) * inv[:, r][:, None]
        acc = acc + jnp.dot(ar, w_ref[D * (r + 1):D * (r + 2), :],
                            preferred_element_type=jnp.float32)
    return jnp.maximum(acc, 0.0)


def _layer_body(h_ref, a_ref, cnt_ref, w_ref, b_ref, out_ref):
    out_ref[...] = _layer_math(h_ref, a_ref, cnt_ref, w_ref, b_ref)


def _tc_layer(h, a4, cnt, w_cat, bias, d_out):
    return pl.pallas_call(
        _layer_body,
        grid=(GRID,),
        in_specs=[
            pl.BlockSpec((NB, D), lambda i: (i, 0)),
            pl.BlockSpec((NC, NB, R, D), lambda i: (0, i, 0, 0)),
            pl.BlockSpec((NC, NB, R, L), lambda i: (0, i, 0, 0)),
            pl.BlockSpec(((R + 1) * D, d_out), lambda i: (0, 0)),
            pl.BlockSpec((1, d_out), lambda i: (0, 0)),
        ],
        out_specs=pl.BlockSpec((NB, d_out), lambda i: (i, 0)),
        out_shape=jax.ShapeDtypeStruct((X_PAD, d_out), jnp.float32),
    )(h, a4, cnt, w_cat, bias)


def _pool_matrix():
    idx = np.arange(D_OUT)
    starts = (idx * N) // D_OUT
    ends = ((idx + 1) * N + D_OUT - 1) // D_OUT
    p = np.zeros((D_OUT, N_PAD), np.float32)
    for b in range(D_OUT):
        p[b, starts[b]:ends[b]] = 1.0 / float(ends[b] - starts[b])
    return jnp.asarray(p)


def _layer2_body(h_ref, a_ref, cnt_ref, w_ref, b_ref, p_ref, out_ref):
    h3 = _layer_math(h_ref, a_ref, cnt_ref, w_ref, b_ref)

    @pl.when(pl.program_id(0) == 0)
    def _():
        out_ref[...] = jnp.zeros_like(out_ref)

    out_ref[...] += jnp.dot(p_ref[...], h3, preferred_element_type=jnp.float32)


def _tc_layer2_pool(h, a4, cnt, w_cat, bias, pmat):
    return pl.pallas_call(
        _layer2_body,
        grid=(GRID,),
        in_specs=[
            pl.BlockSpec((NB, D), lambda i: (i, 0)),
            pl.BlockSpec((NC, NB, R, D), lambda i: (0, i, 0, 0)),
            pl.BlockSpec((NC, NB, R, L), lambda i: (0, i, 0, 0)),
            pl.BlockSpec(((R + 1) * D, D_OUT), lambda i: (0, 0)),
            pl.BlockSpec((1, D_OUT), lambda i: (0, 0)),
            pl.BlockSpec((D_OUT, NB), lambda i: (0, i)),
        ],
        out_specs=pl.BlockSpec((D_OUT, D_OUT), lambda i: (0, 0)),
        out_shape=jax.ShapeDtypeStruct((D_OUT, D_OUT), jnp.float32),
    )(h, a4, cnt, w_cat, bias, pmat)


def kernel(x, edge_index, edge_type, emb, w1, root1, b1, w2, root2, b2):
    src = edge_index[0]
    dst = edge_index[1]
    pad = E_PAD - E
    src_p = jnp.concatenate([src, jnp.zeros((pad,), jnp.int32)])
    dst_p = jnp.concatenate([dst, jnp.full((pad,), N_PAD, jnp.int32)])
    typ_p = jnp.concatenate([edge_type, jnp.zeros((pad,), jnp.int32)])
    x_p = jnp.concatenate([x, jnp.zeros((X_PAD - N,), jnp.int32)])

    w1c = jnp.concatenate([root1, w1.reshape(R * D, D)], axis=0)
    w2c = jnp.concatenate([root2, w2.reshape(R * D, D_OUT)], axis=0)

    h = _emb_gather(emb, x_p)

    a1, cnt = _edge_aggregate(h, src_p, dst_p, typ_p)
    a1 = a1.reshape(NC, AROWS_TOT // R, R, D)
    cnt = cnt.reshape(NC, AROWS_TOT // R, R, L)
    h2 = _tc_layer(h, a1, cnt, w1c, b1.reshape(1, D), D)

    a2, _ = _edge_aggregate(h2, src_p, dst_p, typ_p)
    a2 = a2.reshape(NC, AROWS_TOT // R, R, D)
    return _tc_layer2_pool(h2, a2, cnt, w2c, b2.reshape(1, D_OUT),
                           _pool_matrix())


# speedup vs baseline: 5.3072x; 1.1831x over previous
import numpy as np
import jax
import jax.numpy as jnp
from jax import lax
from jax.experimental import pallas as pl
from jax.experimental.pallas import tpu as pltpu
from jax.experimental.pallas import tpu_sc as plsc

N = 10000
N_PAD = 10240
E = 320000
D = 128
D_OUT = 64
R = 8
VOCAB = 100000

NC, NS, L = 2, 16, 16
NW = NC * NS

EB = 128
NBLK = 79
E_PAD = NW * NBLK * EB
EPT = NBLK * EB

XBLK = 3
X_PAD = NW * XBLK * EB

C_NODES = 864
ROWS = C_NODES * R
NPASS = 12
AROWS_TOT = NPASS * ROWS
ROWS_PT = ROWS // NS
ZROWS = 432
CR_PT = 424

_MESH = plsc.VectorSubcoreMesh(
    core_axis_name="c", subcore_axis_name="s", num_cores=NC, num_subcores=NS
)


def _wid():
    return lax.axis_index("s") * NC + lax.axis_index("c")


def _emb_body(emb_hbm, x_hbm, h_hbm, idx_v, rows_v, sem):
    w = _wid()
    for j in range(XBLK):
        base = (w * XBLK + j) * EB
        pltpu.sync_copy(x_hbm.at[pl.ds(base, EB)], idx_v)
        pltpu.async_copy(emb_hbm.at[idx_v], rows_v, sem).wait()
        pltpu.sync_copy(rows_v, h_hbm.at[pl.ds(base, EB)])


def _emb_gather(emb, x_pad):
    return pl.kernel(
        _emb_body,
        out_type=jax.ShapeDtypeStruct((X_PAD, D), jnp.float32),
        mesh=_MESH,
        compiler_params=pltpu.CompilerParams(
            use_tc_tiling_on_sc=False, needs_layout_passes=False),
        scratch_types=[
            pltpu.VMEM((EB,), jnp.int32),
            pltpu.VMEM((EB, D), jnp.float32),
            pltpu.SemaphoreType.DMA,
        ],
    )(emb, x_pad)


def _agg_impl(with_counts, h_hbm, src_hbm, dst_hbm, typ_hbm, z_hbm, a_hbm,
              e_src, e_key, c_src, c_key, src_row0, src_row1,
              key_row0, key_row1, rows0, rows1, acc_sp, sem0,
              cnt_refs):
    c = lax.axis_index("c")
    s = lax.axis_index("s")
    w = s * NC + c
    if with_counts:
        zc_hbm, cnt_hbm, cnt_v = cnt_refs
    pltpu.sync_copy(src_hbm.at[pl.ds(w * EPT, EPT)], e_src)
    pltpu.sync_copy(dst_hbm.at[pl.ds(w * EPT, EPT)], e_key)
    pltpu.sync_copy(typ_hbm.at[pl.ds(w * EPT, EPT)],
                    c_key.at[pl.ds(0, EPT)])

    def mk(i, _):
        b = i * L
        e_key[pl.ds(b, L)] = e_key[pl.ds(b, L)] * R + c_key[pl.ds(b, L)]
        return 0

    lax.fori_loop(0, EPT // L, mk, 0)

    for p in range(NPASS):
        pltpu.sync_copy(z_hbm, acc_sp.at[pl.ds(s * ROWS_PT, ZROWS)])
        if with_counts:
            pltpu.sync_copy(zc_hbm, cnt_v)
        plsc.subcore_barrier()
        lo = p * ROWS

        ones16 = jnp.ones((L,), jnp.float32)

        def compact(i, off):
            b = i * L
            k = e_key[pl.ds(b, L)] - lo
            sv = e_src[pl.ds(b, L)]
            inr = (k >= 0) & (k < ROWS)
            plsc.store_compressed(c_key.at[pl.ds(off, L)], k, mask=inr)
            plsc.store_compressed(c_src.at[pl.ds(off, L)], sv, mask=inr)
            if with_counts:
                kk = jnp.where(inr, k, ROWS)
                plsc.addupdate_scatter(cnt_v, [kk], ones16)
            return off + jnp.sum(inr.astype(jnp.int32))

        off = lax.fori_loop(0, EPT // L, compact, jnp.int32(0))
        trash_k = jnp.full((L,), ROWS, jnp.int32)
        zero_s = jnp.zeros((L,), jnp.int32)
        for j in range(EB // L):
            c_key[pl.ds(off + j * L, L)] = trash_k
            c_src[pl.ds(off + j * L, L)] = zero_s
        nchunk = (off + EB - 1) // EB

        def prep(q, srow, krow):
            b = q * EB
            for j in range(EB // L):
                srow[pl.ds(j * L, L)] = c_src[pl.ds(b + j * L, L)]
                krow[pl.ds(j * L, L)] = c_key[pl.ds(b + j * L, L)]

        def chunk(q, _):
            prep(q, src_row0, key_row0)
            pltpu.async_copy(h_hbm.at[src_row0], rows0, sem0).wait()
            pltpu.sync_copy(rows0, acc_sp.at[key_row0], add=True)
            return 0

        lax.fori_loop(0, nchunk, chunk, 0)
        plsc.subcore_barrier()
        pltpu.sync_copy(
            acc_sp.at[pl.ds(s * ROWS_PT, ROWS_PT)],
            a_hbm.at[c].at[pl.ds(lo + s * ROWS_PT, ROWS_PT)],
        )
        if with_counts:
            pltpu.sync_copy(
                cnt_v.at[pl.ds(0, ROWS)],
                cnt_hbm.at[c].at[s].at[pl.ds(lo, ROWS)],
            )
        plsc.subcore_barrier()


def _agg_body_cnt(h_hbm, src_hbm, dst_hbm, typ_hbm, z_hbm, zc_hbm,
                  a_hbm, cnt_hbm,
                  e_src, e_key, c_src, c_key, src_row0, src_row1,
                  key_row0, key_row1, rows0, rows1,
                  cnt_v, acc_sp, sem0):
    _agg_impl(True, h_hbm, src_hbm, dst_hbm, typ_hbm, z_hbm, a_hbm,
              e_src, e_key, c_src, c_key, src_row0, src_row1,
              key_row0, key_row1, rows0, rows1, acc_sp, sem0,
              (zc_hbm, cnt_hbm, cnt_v))


def _agg_body_nocnt(h_hbm, src_hbm, dst_hbm, typ_hbm, z_hbm, a_hbm,
                    e_src, e_key, c_src, c_key, src_row0, src_row1,
                    key_row0, key_row1, rows0, rows1, acc_sp, sem0):
    _agg_impl(False, h_hbm, src_hbm, dst_hbm, typ_hbm, z_hbm, a_hbm,
              e_src, e_key, c_src, c_key, src_row0, src_row1,
              key_row0, key_row1, rows0, rows1, acc_sp, sem0, None)


_COMMON_SCRATCH = [
    pltpu.VMEM((EPT,), jnp.int32),
    pltpu.VMEM((EPT,), jnp.int32),
    pltpu.VMEM((EPT + EB,), jnp.int32),
    pltpu.VMEM((EPT + EB,), jnp.int32),
    pltpu.VMEM((EB,), jnp.int32),
    pltpu.VMEM((EB,), jnp.int32),
    pltpu.VMEM((EB,), jnp.int32),
    pltpu.VMEM((EB,), jnp.int32),
    pltpu.VMEM((EB, D), jnp.float32),
    pltpu.VMEM((EB, D), jnp.float32),
]


def _edge_aggregate(h_tab, src_pad, dst_pad, typ_pad, with_counts):
    zrows = jnp.zeros((ZROWS, D), jnp.float32)
    if with_counts:
        zcnt = jnp.zeros((ROWS + 16,), jnp.float32)
        return pl.kernel(
            _agg_body_cnt,
            out_type=(
                jax.ShapeDtypeStruct((NC, AROWS_TOT, D), jnp.float32),
                jax.ShapeDtypeStruct((NC, NS, AROWS_TOT), jnp.float32),
            ),
            mesh=_MESH,
            compiler_params=pltpu.CompilerParams(
                use_tc_tiling_on_sc=False, needs_layout_passes=False),
            scratch_types=_COMMON_SCRATCH + [
                pltpu.VMEM((ROWS + 16,), jnp.float32),
                pltpu.VMEM_SHARED((ROWS + 8, D), jnp.float32),
                pltpu.SemaphoreType.DMA,
            ],
        )(h_tab, src_pad, dst_pad, typ_pad, zrows, zcnt)
    return pl.kernel(
        _agg_body_nocnt,
        out_type=jax.ShapeDtypeStruct((NC, AROWS_TOT, D), jnp.float32),
        mesh=_MESH,
        compiler_params=pltpu.CompilerParams(
            use_tc_tiling_on_sc=False, needs_layout_passes=False),
        scratch_types=_COMMON_SCRATCH + [
            pltpu.VMEM_SHARED((ROWS + 8, D), jnp.float32),
            pltpu.SemaphoreType.DMA,
        ],
    )(h_tab, src_pad, dst_pad, typ_pad, zrows)


NB = 512
GRID = N_PAD // NB


def _layer_math(h_ref, a_ref, cnt_ref, w_ref, b_ref):
    c = jnp.sum(cnt_ref[...], axis=(0, 1))
    inv = 1.0 / jnp.maximum(c, 1.0)
    acc = jnp.dot(h_ref[...], w_ref[0:D, :],
                  preferred_element_type=jnp.float32) + b_ref[0, :][None, :]
    for r in range(R):
        ar = (a_ref[0, :, r, :] + a_ref[1, :, r, :]) * inv[:, r][:, None]
        acc = acc + jnp.dot(ar, w_ref[D * (r + 1):D * (r + 2), :],
                            preferred_element_type=jnp.float32)
    return jnp.maximum(acc, 0.0)


def _layer_body(h_ref, a_ref, cnt_ref, w_ref, b_ref, out_ref):
    out_ref[...] = _layer_math(h_ref, a_ref, cnt_ref, w_ref, b_ref)


def _tc_layer(h, a4, cnt, w_cat, bias, d_out):
    return pl.pallas_call(
        _layer_body,
        grid=(GRID,),
        in_specs=[
            pl.BlockSpec((NB, D), lambda i: (i, 0)),
            pl.BlockSpec((NC, NB, R, D), lambda i: (0, i, 0, 0)),
            pl.BlockSpec((NC, NS, NB, R), lambda i: (0, 0, i, 0)),
            pl.BlockSpec(((R + 1) * D, d_out), lambda i: (0, 0)),
            pl.BlockSpec((1, d_out), lambda i: (0, 0)),
        ],
        out_specs=pl.BlockSpec((NB, d_out), lambda i: (i, 0)),
        out_shape=jax.ShapeDtypeStruct((X_PAD, d_out), jnp.float32),
    )(h, a4, cnt, w_cat, bias)


def _pool_matrix():
    idx = np.arange(D_OUT)
    starts = (idx * N) // D_OUT
    ends = ((idx + 1) * N + D_OUT - 1) // D_OUT
    p = np.zeros((D_OUT, N_PAD), np.float32)
    for b in range(D_OUT):
        p[b, starts[b]:ends[b]] = 1.0 / float(ends[b] - starts[b])
    return jnp.asarray(p)


def _layer2_body(h_ref, a_ref, cnt_ref, w_ref, b_ref, p_ref, out_ref):
    h3 = _layer_math(h_ref, a_ref, cnt_ref, w_ref, b_ref)

    @pl.when(pl.program_id(0) == 0)
    def _():
        out_ref[...] = jnp.zeros_like(out_ref)

    out_ref[...] += jnp.dot(p_ref[...], h3, preferred_element_type=jnp.float32)


def _tc_layer2_pool(h, a4, cnt, w_cat, bias, pmat):
    return pl.pallas_call(
        _layer2_body,
        grid=(GRID,),
        in_specs=[
            pl.BlockSpec((NB, D), lambda i: (i, 0)),
            pl.BlockSpec((NC, NB, R, D), lambda i: (0, i, 0, 0)),
            pl.BlockSpec((NC, NS, NB, R), lambda i: (0, 0, i, 0)),
            pl.BlockSpec(((R + 1) * D, D_OUT), lambda i: (0, 0)),
            pl.BlockSpec((1, D_OUT), lambda i: (0, 0)),
            pl.BlockSpec((D_OUT, NB), lambda i: (0, i)),
        ],
        out_specs=pl.BlockSpec((D_OUT, D_OUT), lambda i: (0, 0)),
        out_shape=jax.ShapeDtypeStruct((D_OUT, D_OUT), jnp.float32),
    )(h, a4, cnt, w_cat, bias, pmat)


def kernel(x, edge_index, edge_type, emb, w1, root1, b1, w2, root2, b2):
    src = edge_index[0]
    dst = edge_index[1]
    pad = E_PAD - E
    src_p = jnp.concatenate([src, jnp.zeros((pad,), jnp.int32)])
    dst_p = jnp.concatenate([dst, jnp.full((pad,), N_PAD, jnp.int32)])
    typ_p = jnp.concatenate([edge_type, jnp.zeros((pad,), jnp.int32)])
    x_p = jnp.concatenate([x, jnp.zeros((X_PAD - N,), jnp.int32)])

    w1c = jnp.concatenate([root1, w1.reshape(R * D, D)], axis=0)
    w2c = jnp.concatenate([root2, w2.reshape(R * D, D_OUT)], axis=0)

    h = _emb_gather(emb, x_p)

    a1, cnt = _edge_aggregate(h, src_p, dst_p, typ_p, True)
    a1 = a1.reshape(NC, AROWS_TOT // R, R, D)
    cnt = cnt.reshape(NC, NS, AROWS_TOT // R, R)
    h2 = _tc_layer(h, a1, cnt, w1c, b1.reshape(1, D), D)

    a2 = _edge_aggregate(h2, src_p, dst_p, typ_p, False)
    a2 = a2.reshape(NC, AROWS_TOT // R, R, D)
    return _tc_layer2_pool(h2, a2, cnt, w2c, b2.reshape(1, D_OUT),
                           _pool_matrix())
